# Initial kernel scaffold; baseline (speedup 1.0000x reference)
import jax
import jax.numpy as jnp
from jax.experimental import pallas as pl


def kernel(x, edge_index, W_in, b_in, coeffs0, W_out):
    raise NotImplementedError("write your pallas kernel here")


# speedup vs baseline: 5.0970x; 5.0970x over previous
import functools

import jax
import jax.numpy as jnp
from jax import lax
from jax.experimental import pallas as pl
from jax.experimental.pallas import tpu as pltpu
from jax.experimental.pallas import tpu_sc as plsc

N = 10000
E = 320000
IN_FEAT = 128
HIDDEN = 64
OUT_FEAT = 64
GRID = 4

NC = 2
NS = 16
NTILES = NC * NS
BATCH = 128
NB = 79
E_PAD = NTILES * NB * BATCH
NP = 10240

ROW_BLK = 2000


def _lin_in_body(x_ref, w_ref, b_ref, o_ref):
    acc = lax.dot_general(
        x_ref[...], w_ref[...], (((1,), (1,)), ((), ())),
        preferred_element_type=jnp.float32)
    o_ref[...] = acc + b_ref[...]


def _lin_in(x, W_in, b_in):
    return pl.pallas_call(
        _lin_in_body,
        grid=(N // ROW_BLK,),
        in_specs=[
            pl.BlockSpec((ROW_BLK, IN_FEAT), lambda i: (i, 0)),
            pl.BlockSpec((HIDDEN, IN_FEAT), lambda i: (0, 0)),
            pl.BlockSpec((1, HIDDEN), lambda i: (0, 0)),
        ],
        out_specs=pl.BlockSpec((ROW_BLK, HIDDEN), lambda i: (i, 0)),
        out_shape=jax.ShapeDtypeStruct((N, HIDDEN), jnp.float32),
    )(x, W_in, b_in.reshape(1, HIDDEN))


def _sc_spmm_body(rows_hbm, cols_hbm, h_hbm, z_hbm, out_hbm,
                  colv, rowv, rbuf, acc, sem):
    c = lax.axis_index("c")
    s = lax.axis_index("s")
    wid = c * NS + s
    rpt = NP // NS

    pltpu.sync_copy(z_hbm, acc.at[pl.ds(s * rpt, rpt)])
    pltpu.sync_copy(cols_hbm.at[wid], colv)
    pltpu.sync_copy(rows_hbm.at[wid], rowv)
    plsc.subcore_barrier()

    def body(j, carry):
        pltpu.async_copy(h_hbm.at[colv.at[j]], rbuf, sem).wait()
        pltpu.sync_copy(rbuf, acc.at[rowv.at[j]], add=True)
        return carry

    lax.fori_loop(0, NB, body, 0)
    plsc.subcore_barrier()

    pltpu.sync_copy(acc.at[pl.ds(s * rpt, rpt)],
                    out_hbm.at[c, pl.ds(s * rpt, rpt)])


_sc_spmm = functools.partial(
    pl.kernel,
    out_type=jax.ShapeDtypeStruct((NC, NP, HIDDEN), jnp.float32),
    mesh=plsc.VectorSubcoreMesh(
        core_axis_name="c", subcore_axis_name="s",
        num_cores=NC, num_subcores=NS),
    scratch_types=[
        pltpu.VMEM((NB, BATCH), jnp.int32),
        pltpu.VMEM((NB, BATCH), jnp.int32),
        pltpu.VMEM((BATCH, HIDDEN), jnp.float32),
        pltpu.VMEM_SHARED((NP, HIDDEN), jnp.float32),
        pltpu.SemaphoreType.DMA,
    ],
    compiler_params=pltpu.CompilerParams(use_tc_tiling_on_sc=False),
)(_sc_spmm_body)


def _post_body(p0_ref, p1_ref, wf_ref, wo_ref, o_ref):
    a = p0_ref[...] + p1_ref[...]
    feats = []
    for g in range(GRID):
        feats.append(jnp.cos((g + 1.0) * a))
    for g in range(GRID):
        feats.append(jnp.sin((g + 1.0) * a))
    feat = jnp.concatenate(feats, axis=1)
    y = jnp.dot(feat, wf_ref[...], preferred_element_type=jnp.float32)
    o = lax.dot_general(
        y, wo_ref[...], (((1,), (1,)), ((), ())),
        preferred_element_type=jnp.float32)
    m = jnp.max(o, axis=-1, keepdims=True)
    ex = jnp.exp(o - m)
    o_ref[...] = (o - m) - jnp.log(jnp.sum(ex, axis=-1, keepdims=True))


def _post(p0, p1, WF, W_out):
    F = 2 * GRID * HIDDEN
    return pl.pallas_call(
        _post_body,
        grid=(N // ROW_BLK,),
        in_specs=[
            pl.BlockSpec((ROW_BLK, HIDDEN), lambda i: (i, 0)),
            pl.BlockSpec((ROW_BLK, HIDDEN), lambda i: (i, 0)),
            pl.BlockSpec((F, HIDDEN), lambda i: (0, 0)),
            pl.BlockSpec((OUT_FEAT, HIDDEN), lambda i: (0, 0)),
        ],
        out_specs=pl.BlockSpec((ROW_BLK, OUT_FEAT), lambda i: (i, 0)),
        out_shape=jax.ShapeDtypeStruct((N, OUT_FEAT), jnp.float32),
    )(p0, p1, WF, W_out)


def kernel(x, edge_index, W_in, b_in, coeffs0, W_out):
    h = _lin_in(x, W_in, b_in)

    row = edge_index[0]
    col = edge_index[1]
    pad = E_PAD - E
    rowp = jnp.concatenate(
        [row, jnp.full((pad,), N, jnp.int32)]).reshape(NTILES, NB, BATCH)
    colp = jnp.concatenate(
        [col, jnp.zeros((pad,), jnp.int32)]).reshape(NTILES, NB, BATCH)
    zeros = jnp.zeros((NP // NS, HIDDEN), jnp.float32)

    partials = _sc_spmm(rowp, colp, h, zeros)

    WFc = jnp.transpose(coeffs0[0], (2, 1, 0)).reshape(GRID * HIDDEN, HIDDEN)
    WFs = jnp.transpose(coeffs0[1], (2, 1, 0)).reshape(GRID * HIDDEN, HIDDEN)
    WF = jnp.concatenate([WFc, WFs], axis=0)

    return _post(partials[0, :N], partials[1, :N], WF, W_out)
